# baseline (speedup 1.0000x reference)
import functools

import jax
import jax.numpy as jnp
from jax import lax
from jax.experimental import pallas as pl
from jax.experimental.pallas import tpu as pltpu
from jax.experimental.pallas import tpu_sc as plsc

A = 128
NBR = 16
N = 10000
M = 32
EPS = 1e-5

E = N * M
_NC = 2
_NS = 16
_NW = _NC * _NS
_IDXW = 128
_ROWS = -(-E // _IDXW)
_RPW = (-(-_ROWS // _NW) + 7) // 8 * 8
_ROWS_PAD = _RPW * _NW
_E_PAD = _ROWS_PAD * _IDXW

_T = 200
_TE = _T * M
_GRID = N // _T
_T3 = 2000


_NB = 6
_KL = 3


def _sc_gather(table, idx2d):
    mesh = plsc.VectorSubcoreMesh(core_axis_name="c", subcore_axis_name="s")

    @functools.partial(
        pl.kernel,
        out_type=jax.ShapeDtypeStruct((_ROWS_PAD, _IDXW, A), jnp.int32),
        mesh=mesh,
        scratch_types=[
            pltpu.VMEM((_RPW, _IDXW), jnp.int32),
            pltpu.VMEM((_NB * _IDXW, A), jnp.int32),
            pltpu.SemaphoreType.DMA((_NB,)),
        ],
    )
    def k(table_hbm, idx_hbm, out_hbm, idx_v, bufs, sems):
        wid = lax.axis_index("s") * _NC + lax.axis_index("c")
        rbase = wid * _RPW
        pltpu.sync_copy(idx_hbm.at[pl.ds(rbase, _RPW)], idx_v)

        def body(jj, carry):
            b = lax.rem(jj, _NB)
            buf_b = bufs.at[pl.ds(b * _IDXW, _IDXW)]

            @pl.when(jj >= _NB)
            def _():
                pltpu.make_async_copy(
                    buf_b, out_hbm.at[rbase + jj - _NB], sems.at[b]).wait()

            @pl.when(jj < _RPW)
            def _():
                pltpu.async_copy(
                    table_hbm.at[idx_v.at[jj]], buf_b, sems.at[b])

            j2 = jj - _KL
            b2 = lax.rem(j2 + _NB, _NB)
            buf_b2 = bufs.at[pl.ds(b2 * _IDXW, _IDXW)]

            @pl.when((jj >= _KL) & (j2 < _RPW))
            def _():
                pltpu.make_async_copy(
                    table_hbm.at[idx_v.at[0]], buf_b2, sems.at[b2]).wait()
                pltpu.async_copy(buf_b2, out_hbm.at[rbase + j2], sems.at[b2])

            return carry

        lax.fori_loop(0, _RPW + _KL, body, 0)

        for c in range(_RPW - _NB + _KL, _RPW):
            b = c % _NB
            pltpu.make_async_copy(
                bufs.at[pl.ds(b * _IDXW, _IDXW)],
                out_hbm.at[rbase + c], sems.at[b]).wait()

    return k(table, idx2d)


_TG = 2000


def _g_body(atom_ref, wn_ref, out_ref):
    g = jnp.dot(atom_ref[...], wn_ref[...],
                preferred_element_type=jnp.float32)
    g_bf = g.astype(jnp.bfloat16)
    inter = jnp.stack([g_bf[:, :A], g_bf[:, A:]], axis=1)
    out_ref[...] = pltpu.bitcast(inter.reshape(2 * _TG, A), jnp.int32)


def _p1_body(staged_ref, nbr_ref, atom_ref, ws_ref, we_ref,
             b_ref, out_ref):
    i = pl.program_id(0)
    xg = pltpu.bitcast(staged_ref[...], jnp.bfloat16).reshape(_TE, 2, A)
    g32 = jnp.concatenate([xg[:, 0, :], xg[:, 1, :]], axis=1).astype(jnp.float32)
    z = g32 + jnp.dot(nbr_ref[...], we_ref[...], preferred_element_type=jnp.float32)
    s = jnp.dot(atom_ref[...], ws_ref[...], preferred_element_type=jnp.float32) + b_ref[...]
    z3 = z.reshape(_T, M, 2 * A) + s[:, None, :]

    @pl.when(i == 0)
    def _():
        out_ref[...] = jnp.zeros_like(out_ref)

    out_ref[0:1, :] += jnp.sum(z3, axis=(0, 1))[None, :]
    out_ref[1:2, :] += jnp.sum(z3 * z3, axis=(0, 1))[None, :]


def _p2_body(sums_ref, g1_ref, b1_ref, staged_ref, nbr_ref, atom_ref,
             ws_ref, we_ref, b_ref, ns_ref, st2_ref):
    i = pl.program_id(0)
    nm = jnp.float32(E)
    mean = sums_ref[0:1, :] / nm
    var = sums_ref[1:2, :] / nm - mean * mean
    a = g1_ref[...] * lax.rsqrt(var + EPS)
    d = b1_ref[...] - mean * a

    xg = pltpu.bitcast(staged_ref[...], jnp.bfloat16).reshape(_TE, 2, A)
    g32 = jnp.concatenate([xg[:, 0, :], xg[:, 1, :]], axis=1).astype(jnp.float32)
    z = g32 + jnp.dot(nbr_ref[...], we_ref[...], preferred_element_type=jnp.float32)
    s = jnp.dot(atom_ref[...], ws_ref[...], preferred_element_type=jnp.float32) + b_ref[...]
    z3 = z.reshape(_T, M, 2 * A) + s[:, None, :]
    zt = z3 * a[0][None, None, :] + d[0][None, None, :]

    f = zt[:, :, :A]
    c = zt[:, :, A:]
    p = (1.0 / (1.0 + jnp.exp(-f))) * jnp.maximum(c, 0.0)
    ns = jnp.sum(p, axis=1)
    ns_ref[...] = ns

    @pl.when(i == 0)
    def _():
        st2_ref[...] = jnp.zeros_like(st2_ref)

    st2_ref[0:1, :] += jnp.sum(ns, axis=0)[None, :]
    st2_ref[1:2, :] += jnp.sum(ns * ns, axis=0)[None, :]


def _p3_body(st2_ref, g2_ref, b2_ref, atom_ref, ns_ref, out_ref):
    nn = jnp.float32(N)
    mean = st2_ref[0:1, :] / nn
    var = st2_ref[1:2, :] / nn - mean * mean
    a = g2_ref[...] * lax.rsqrt(var + EPS)
    d = b2_ref[...] - mean * a
    out_ref[...] = jnp.maximum(atom_ref[...] + ns_ref[...] * a + d, 0.0)


def kernel(atom_in_fea, nbr_fea, nbr_fea_idx, W_full, b_full,
           bn1_gamma, bn1_beta, bn2_gamma, bn2_beta):
    atom_in_fea = atom_in_fea.astype(jnp.float32)
    idx = nbr_fea_idx.astype(jnp.int32).reshape(-1)
    idx2d = jnp.pad(idx, (0, _E_PAD - E)).reshape(_ROWS_PAD, _IDXW)

    wn = W_full[A:2 * A].astype(jnp.bfloat16)
    gpacked = pl.pallas_call(
        _g_body,
        grid=(N // _TG,),
        in_specs=[
            pl.BlockSpec((_TG, A), lambda i: (i, 0)),
            pl.BlockSpec((A, 2 * A), lambda i: (0, 0)),
        ],
        out_specs=pl.BlockSpec((_TG, A), lambda i: (i, 0)),
        out_shape=jax.ShapeDtypeStruct((N, A), jnp.int32),
    )(atom_in_fea, wn)

    staged = _sc_gather(gpacked, idx2d).reshape(_E_PAD, A)
    nbr2 = nbr_fea.astype(jnp.float32).reshape(E, NBR)

    ws = W_full[:A]
    we = W_full[2 * A:]
    b2d = b_full.reshape(1, 2 * A)
    g1 = bn1_gamma.reshape(1, 2 * A)
    be1 = bn1_beta.reshape(1, 2 * A)
    g2 = bn2_gamma.reshape(1, A)
    be2 = bn2_beta.reshape(1, A)

    edge_specs = [
        pl.BlockSpec((_TE, A), lambda i: (i, 0)),
        pl.BlockSpec((_TE, NBR), lambda i: (i, 0)),
        pl.BlockSpec((_T, A), lambda i: (i, 0)),
        pl.BlockSpec((A, 2 * A), lambda i: (0, 0)),
        pl.BlockSpec((NBR, 2 * A), lambda i: (0, 0)),
        pl.BlockSpec((1, 2 * A), lambda i: (0, 0)),
    ]

    sums = pl.pallas_call(
        _p1_body,
        grid=(_GRID,),
        in_specs=edge_specs,
        out_specs=pl.BlockSpec((8, 2 * A), lambda i: (0, 0)),
        out_shape=jax.ShapeDtypeStruct((8, 2 * A), jnp.float32),
        compiler_params=pltpu.CompilerParams(
            dimension_semantics=("arbitrary",)),
    )(staged, nbr2, atom_in_fea, ws, we, b2d)

    small = [
        pl.BlockSpec((8, 2 * A), lambda i: (0, 0)),
        pl.BlockSpec((1, 2 * A), lambda i: (0, 0)),
        pl.BlockSpec((1, 2 * A), lambda i: (0, 0)),
    ]
    ns, st2 = pl.pallas_call(
        _p2_body,
        grid=(_GRID,),
        in_specs=small + edge_specs,
        out_specs=[
            pl.BlockSpec((_T, A), lambda i: (i, 0)),
            pl.BlockSpec((8, A), lambda i: (0, 0)),
        ],
        out_shape=[
            jax.ShapeDtypeStruct((N, A), jnp.float32),
            jax.ShapeDtypeStruct((8, A), jnp.float32),
        ],
        compiler_params=pltpu.CompilerParams(
            dimension_semantics=("arbitrary",)),
    )(sums, g1, be1, staged, nbr2, atom_in_fea, ws, we, b2d)

    out = pl.pallas_call(
        _p3_body,
        grid=(N // _T3,),
        in_specs=[
            pl.BlockSpec((8, A), lambda i: (0, 0)),
            pl.BlockSpec((1, A), lambda i: (0, 0)),
            pl.BlockSpec((1, A), lambda i: (0, 0)),
            pl.BlockSpec((_T3, A), lambda i: (i, 0)),
            pl.BlockSpec((_T3, A), lambda i: (i, 0)),
        ],
        out_specs=pl.BlockSpec((_T3, A), lambda i: (i, 0)),
        out_shape=jax.ShapeDtypeStruct((N, A), jnp.float32),
    )(st2, g2, be2, atom_in_fea, ns)

    return out


# speedup vs baseline: 1.4992x; 1.4992x over previous
import functools

import jax
import jax.numpy as jnp
from jax import lax
from jax.experimental import pallas as pl
from jax.experimental.pallas import tpu as pltpu
from jax.experimental.pallas import tpu_sc as plsc

A = 128
NBR = 16
N = 10000
M = 32
EPS = 1e-5

E = N * M
_NC = 2
_NS = 16
_IDXW = 128
_ROWS_PAD = 2560
_E_PAD = _ROWS_PAD * _IDXW
_RPW0 = 128
_RPW1 = 32

_T = 200
_TE = _T * M
_GRID = N // _T
_T3 = 2000

_NB = 6
_KL = 3


def _sc_gather(table, idx2d):
    mesh = plsc.VectorSubcoreMesh(core_axis_name="c", subcore_axis_name="s")

    @functools.partial(
        pl.kernel,
        out_type=jax.ShapeDtypeStruct((_ROWS_PAD, _IDXW, A), jnp.float32),
        mesh=mesh,
        scratch_types=[
            pltpu.VMEM((_RPW0, _IDXW), jnp.int32),
            pltpu.VMEM((_NB * _IDXW, A), jnp.float32),
            pltpu.SemaphoreType.DMA((_NB,)),
        ],
    )
    def k(table_hbm, idx_hbm, out_hbm, idx_v, bufs, sems):
        cid = lax.axis_index("c")
        sid = lax.axis_index("s")
        rbase = jnp.where(cid == 0, sid * _RPW0, _NS * _RPW0 + sid * _RPW1)
        nrows = jnp.where(cid == 0, _RPW0, _RPW1)

        @pl.when(cid == 0)
        def _():
            pltpu.sync_copy(idx_hbm.at[pl.ds(rbase, _RPW0)], idx_v)

        @pl.when(cid != 0)
        def _():
            pltpu.sync_copy(idx_hbm.at[pl.ds(rbase, _RPW1)],
                            idx_v.at[pl.ds(0, _RPW1)])

        def body(jj, carry):
            b = lax.rem(jj, _NB)
            buf_b = bufs.at[pl.ds(b * _IDXW, _IDXW)]

            @pl.when((jj >= _NB) & (jj - _NB < nrows - (_NB - _KL)))
            def _():
                pltpu.make_async_copy(
                    buf_b, out_hbm.at[rbase + jj - _NB], sems.at[b]).wait()

            @pl.when(jj < nrows)
            def _():
                pltpu.async_copy(
                    table_hbm.at[idx_v.at[jj]], buf_b, sems.at[b])

            j2 = jj - _KL
            b2 = lax.rem(j2 + _NB, _NB)
            buf_b2 = bufs.at[pl.ds(b2 * _IDXW, _IDXW)]

            @pl.when((jj >= _KL) & (j2 < nrows))
            def _():
                pltpu.make_async_copy(
                    table_hbm.at[idx_v.at[0]], buf_b2, sems.at[b2]).wait()
                pltpu.async_copy(buf_b2, out_hbm.at[rbase + j2], sems.at[b2])

            return carry

        lax.fori_loop(0, _RPW0 + _KL, body, 0)

        def drain(t, carry):
            c2 = nrows - (_NB - _KL) + t
            b = lax.rem(c2, _NB)
            pltpu.make_async_copy(
                bufs.at[pl.ds(b * _IDXW, _IDXW)],
                out_hbm.at[rbase + c2], sems.at[b]).wait()
            return carry

        lax.fori_loop(0, _NB - _KL, drain, 0)

    return k(table, idx2d)


def _p1_body(staged_ref, nbr_ref, atom_ref, ws_ref, wn_ref, we_ref, b_ref,
             out_ref):
    i = pl.program_id(0)
    xg = staged_ref[...].astype(jnp.bfloat16)
    z = (jnp.dot(xg, wn_ref[...], preferred_element_type=jnp.float32)
         + jnp.dot(nbr_ref[...], we_ref[...], preferred_element_type=jnp.float32))
    s = jnp.dot(atom_ref[...], ws_ref[...], preferred_element_type=jnp.float32) + b_ref[...]
    z3 = z.reshape(_T, M, 2 * A) + s[:, None, :]

    @pl.when(i == 0)
    def _():
        out_ref[...] = jnp.zeros_like(out_ref)

    out_ref[0:1, :] += jnp.sum(z3, axis=(0, 1))[None, :]
    out_ref[1:2, :] += jnp.sum(z3 * z3, axis=(0, 1))[None, :]


def _p2_body(sums_ref, g1_ref, b1_ref, staged_ref, nbr_ref, atom_ref,
             ws_ref, wn_ref, we_ref, b_ref, ns_ref, st2_ref):
    i = pl.program_id(0)
    nm = jnp.float32(E)
    mean = sums_ref[0:1, :] / nm
    var = sums_ref[1:2, :] / nm - mean * mean
    a = g1_ref[...] * lax.rsqrt(var + EPS)
    d = b1_ref[...] - mean * a

    xg = staged_ref[...].astype(jnp.bfloat16)
    z = (jnp.dot(xg, wn_ref[...], preferred_element_type=jnp.float32)
         + jnp.dot(nbr_ref[...], we_ref[...], preferred_element_type=jnp.float32))
    s = jnp.dot(atom_ref[...], ws_ref[...], preferred_element_type=jnp.float32) + b_ref[...]
    z3 = z.reshape(_T, M, 2 * A) + s[:, None, :]
    zt = z3 * a[0][None, None, :] + d[0][None, None, :]

    f = zt[:, :, :A]
    c = zt[:, :, A:]
    p = (1.0 / (1.0 + jnp.exp(-f))) * jnp.maximum(c, 0.0)
    ns = jnp.sum(p, axis=1)
    ns_ref[...] = ns

    @pl.when(i == 0)
    def _():
        st2_ref[...] = jnp.zeros_like(st2_ref)

    st2_ref[0:1, :] += jnp.sum(ns, axis=0)[None, :]
    st2_ref[1:2, :] += jnp.sum(ns * ns, axis=0)[None, :]


def _p3_body(st2_ref, g2_ref, b2_ref, atom_ref, ns_ref, out_ref):
    nn = jnp.float32(N)
    mean = st2_ref[0:1, :] / nn
    var = st2_ref[1:2, :] / nn - mean * mean
    a = g2_ref[...] * lax.rsqrt(var + EPS)
    d = b2_ref[...] - mean * a
    out_ref[...] = jnp.maximum(atom_ref[...] + ns_ref[...] * a + d, 0.0)


def kernel(atom_in_fea, nbr_fea, nbr_fea_idx, W_full, b_full,
           bn1_gamma, bn1_beta, bn2_gamma, bn2_beta):
    atom_in_fea = atom_in_fea.astype(jnp.float32)
    idx = nbr_fea_idx.astype(jnp.int32).reshape(-1)
    idx2d = jnp.pad(idx, (0, _E_PAD - E)).reshape(_ROWS_PAD, _IDXW)

    staged = _sc_gather(atom_in_fea, idx2d).reshape(_E_PAD, A)
    nbr2 = nbr_fea.astype(jnp.float32).reshape(E, NBR)

    ws = W_full[:A]
    wn = W_full[A:2 * A].astype(jnp.bfloat16)
    we = W_full[2 * A:]
    b2d = b_full.reshape(1, 2 * A)
    g1 = bn1_gamma.reshape(1, 2 * A)
    be1 = bn1_beta.reshape(1, 2 * A)
    g2 = bn2_gamma.reshape(1, A)
    be2 = bn2_beta.reshape(1, A)

    edge_specs = [
        pl.BlockSpec((_TE, A), lambda i: (i, 0)),
        pl.BlockSpec((_TE, NBR), lambda i: (i, 0)),
        pl.BlockSpec((_T, A), lambda i: (i, 0)),
        pl.BlockSpec((A, 2 * A), lambda i: (0, 0)),
        pl.BlockSpec((A, 2 * A), lambda i: (0, 0)),
        pl.BlockSpec((NBR, 2 * A), lambda i: (0, 0)),
        pl.BlockSpec((1, 2 * A), lambda i: (0, 0)),
    ]

    sums = pl.pallas_call(
        _p1_body,
        grid=(_GRID,),
        in_specs=edge_specs,
        out_specs=pl.BlockSpec((8, 2 * A), lambda i: (0, 0)),
        out_shape=jax.ShapeDtypeStruct((8, 2 * A), jnp.float32),
        compiler_params=pltpu.CompilerParams(
            dimension_semantics=("arbitrary",)),
    )(staged, nbr2, atom_in_fea, ws, wn, we, b2d)

    small = [
        pl.BlockSpec((8, 2 * A), lambda i: (0, 0)),
        pl.BlockSpec((1, 2 * A), lambda i: (0, 0)),
        pl.BlockSpec((1, 2 * A), lambda i: (0, 0)),
    ]
    ns, st2 = pl.pallas_call(
        _p2_body,
        grid=(_GRID,),
        in_specs=small + edge_specs,
        out_specs=[
            pl.BlockSpec((_T, A), lambda i: (i, 0)),
            pl.BlockSpec((8, A), lambda i: (0, 0)),
        ],
        out_shape=[
            jax.ShapeDtypeStruct((N, A), jnp.float32),
            jax.ShapeDtypeStruct((8, A), jnp.float32),
        ],
        compiler_params=pltpu.CompilerParams(
            dimension_semantics=("arbitrary",)),
    )(sums, g1, be1, staged, nbr2, atom_in_fea, ws, wn, we, b2d)

    out = pl.pallas_call(
        _p3_body,
        grid=(N // _T3,),
        in_specs=[
            pl.BlockSpec((8, A), lambda i: (0, 0)),
            pl.BlockSpec((1, A), lambda i: (0, 0)),
            pl.BlockSpec((1, A), lambda i: (0, 0)),
            pl.BlockSpec((_T3, A), lambda i: (i, 0)),
            pl.BlockSpec((_T3, A), lambda i: (i, 0)),
        ],
        out_specs=pl.BlockSpec((_T3, A), lambda i: (i, 0)),
        out_shape=jax.ShapeDtypeStruct((N, A), jnp.float32),
    )(st2, g2, be2, atom_in_fea, ns)

    return out


# speedup vs baseline: 2.1765x; 1.4518x over previous
import functools

import jax
import jax.numpy as jnp
from jax import lax
from jax.experimental import pallas as pl
from jax.experimental.pallas import tpu as pltpu
from jax.experimental.pallas import tpu_sc as plsc

A = 128
NBR = 16
N = 10000
M = 32
EPS = 1e-5

E = N * M
_NS = 16
_CW = 100
_ROWS = E // _CW
_RPW = _ROWS // _NS

_T = 200
_TE = _T * M
_GRID = N // _T
_T3 = 2000

_NB = 6
_KL = 3


def _sc_gather(table, idx2d):
    mesh = plsc.VectorSubcoreMesh(core_axis_name="c", subcore_axis_name="s",
                                  num_cores=1)

    @functools.partial(
        pl.kernel,
        out_type=jax.ShapeDtypeStruct((_ROWS, _CW, A), jnp.float32),
        mesh=mesh,
        scratch_types=[
            pltpu.VMEM((_RPW, _CW), jnp.int32),
            pltpu.VMEM((_NB * _CW, A), jnp.float32),
            pltpu.SemaphoreType.DMA((_NB,)),
        ],
    )
    def k(table_hbm, idx_hbm, out_hbm, idx_v, bufs, sems):
        sid = lax.axis_index("s")
        rbase = sid * _RPW
        pltpu.sync_copy(idx_hbm.at[pl.ds(rbase, _RPW)], idx_v)

        def body(jj, carry):
            b = lax.rem(jj, _NB)
            buf_b = bufs.at[pl.ds(b * _CW, _CW)]

            @pl.when((jj >= _NB) & (jj - _NB < _RPW - (_NB - _KL)))
            def _():
                pltpu.make_async_copy(
                    buf_b, out_hbm.at[rbase + jj - _NB], sems.at[b]).wait()

            @pl.when(jj < _RPW)
            def _():
                pltpu.async_copy(
                    table_hbm.at[idx_v.at[jj]], buf_b, sems.at[b])

            j2 = jj - _KL
            b2 = lax.rem(j2 + _NB, _NB)
            buf_b2 = bufs.at[pl.ds(b2 * _CW, _CW)]

            @pl.when((jj >= _KL) & (j2 < _RPW))
            def _():
                pltpu.make_async_copy(
                    table_hbm.at[idx_v.at[0]], buf_b2, sems.at[b2]).wait()
                pltpu.async_copy(buf_b2, out_hbm.at[rbase + j2], sems.at[b2])

            return carry

        lax.fori_loop(0, _RPW + _KL, body, 0)

        for c in range(_RPW - (_NB - _KL), _RPW):
            b = c % _NB
            pltpu.make_async_copy(
                bufs.at[pl.ds(b * _CW, _CW)],
                out_hbm.at[rbase + c], sems.at[b]).wait()

    return k(table, idx2d)


def _p1_body(staged_ref, nbr_ref, atom_ref, ws_ref, wn_ref, we_ref, b_ref,
             out_ref):
    i = pl.program_id(0)
    xg = staged_ref[...].astype(jnp.bfloat16)
    z = (jnp.dot(xg, wn_ref[...], preferred_element_type=jnp.float32)
         + jnp.dot(nbr_ref[...], we_ref[...], preferred_element_type=jnp.float32))
    s = jnp.dot(atom_ref[...], ws_ref[...], preferred_element_type=jnp.float32) + b_ref[...]
    z3 = z.reshape(_T, M, 2 * A) + s[:, None, :]

    @pl.when(i == 0)
    def _():
        out_ref[...] = jnp.zeros_like(out_ref)

    out_ref[0:1, :] += jnp.sum(z3, axis=(0, 1))[None, :]
    out_ref[1:2, :] += jnp.sum(z3 * z3, axis=(0, 1))[None, :]


def _p2_body(sums_ref, g1_ref, b1_ref, staged_ref, nbr_ref, atom_ref,
             ws_ref, wn_ref, we_ref, b_ref, ns_ref, st2_ref):
    i = pl.program_id(0)
    nm = jnp.float32(E)
    mean = sums_ref[0:1, :] / nm
    var = sums_ref[1:2, :] / nm - mean * mean
    a = g1_ref[...] * lax.rsqrt(var + EPS)
    d = b1_ref[...] - mean * a

    xg = staged_ref[...].astype(jnp.bfloat16)
    z = (jnp.dot(xg, wn_ref[...], preferred_element_type=jnp.float32)
         + jnp.dot(nbr_ref[...], we_ref[...], preferred_element_type=jnp.float32))
    s = jnp.dot(atom_ref[...], ws_ref[...], preferred_element_type=jnp.float32) + b_ref[...]
    z3 = z.reshape(_T, M, 2 * A) + s[:, None, :]
    zt = z3 * a[0][None, None, :] + d[0][None, None, :]

    f = zt[:, :, :A]
    c = zt[:, :, A:]
    p = (1.0 / (1.0 + jnp.exp(-f))) * jnp.maximum(c, 0.0)
    ns = jnp.sum(p, axis=1)
    ns_ref[...] = ns

    @pl.when(i == 0)
    def _():
        st2_ref[...] = jnp.zeros_like(st2_ref)

    st2_ref[0:1, :] += jnp.sum(ns, axis=0)[None, :]
    st2_ref[1:2, :] += jnp.sum(ns * ns, axis=0)[None, :]


def _p3_body(st2_ref, g2_ref, b2_ref, atom_ref, ns_ref, out_ref):
    nn = jnp.float32(N)
    mean = st2_ref[0:1, :] / nn
    var = st2_ref[1:2, :] / nn - mean * mean
    a = g2_ref[...] * lax.rsqrt(var + EPS)
    d = b2_ref[...] - mean * a
    out_ref[...] = jnp.maximum(atom_ref[...] + ns_ref[...] * a + d, 0.0)


def kernel(atom_in_fea, nbr_fea, nbr_fea_idx, W_full, b_full,
           bn1_gamma, bn1_beta, bn2_gamma, bn2_beta):
    atom_in_fea = atom_in_fea.astype(jnp.float32)
    idx2d = nbr_fea_idx.astype(jnp.int32).reshape(_ROWS, _CW)

    staged = _sc_gather(atom_in_fea, idx2d).reshape(E, A)
    nbr2 = nbr_fea.astype(jnp.float32).reshape(E, NBR)

    ws = W_full[:A]
    wn = W_full[A:2 * A].astype(jnp.bfloat16)
    we = W_full[2 * A:]
    b2d = b_full.reshape(1, 2 * A)
    g1 = bn1_gamma.reshape(1, 2 * A)
    be1 = bn1_beta.reshape(1, 2 * A)
    g2 = bn2_gamma.reshape(1, A)
    be2 = bn2_beta.reshape(1, A)

    edge_specs = [
        pl.BlockSpec((_TE, A), lambda i: (i, 0)),
        pl.BlockSpec((_TE, NBR), lambda i: (i, 0)),
        pl.BlockSpec((_T, A), lambda i: (i, 0)),
        pl.BlockSpec((A, 2 * A), lambda i: (0, 0)),
        pl.BlockSpec((A, 2 * A), lambda i: (0, 0)),
        pl.BlockSpec((NBR, 2 * A), lambda i: (0, 0)),
        pl.BlockSpec((1, 2 * A), lambda i: (0, 0)),
    ]

    sums = pl.pallas_call(
        _p1_body,
        grid=(_GRID,),
        in_specs=edge_specs,
        out_specs=pl.BlockSpec((8, 2 * A), lambda i: (0, 0)),
        out_shape=jax.ShapeDtypeStruct((8, 2 * A), jnp.float32),
        compiler_params=pltpu.CompilerParams(
            dimension_semantics=("arbitrary",)),
    )(staged, nbr2, atom_in_fea, ws, wn, we, b2d)

    small = [
        pl.BlockSpec((8, 2 * A), lambda i: (0, 0)),
        pl.BlockSpec((1, 2 * A), lambda i: (0, 0)),
        pl.BlockSpec((1, 2 * A), lambda i: (0, 0)),
    ]
    ns, st2 = pl.pallas_call(
        _p2_body,
        grid=(_GRID,),
        in_specs=small + edge_specs,
        out_specs=[
            pl.BlockSpec((_T, A), lambda i: (i, 0)),
            pl.BlockSpec((8, A), lambda i: (0, 0)),
        ],
        out_shape=[
            jax.ShapeDtypeStruct((N, A), jnp.float32),
            jax.ShapeDtypeStruct((8, A), jnp.float32),
        ],
        compiler_params=pltpu.CompilerParams(
            dimension_semantics=("arbitrary",)),
    )(sums, g1, be1, staged, nbr2, atom_in_fea, ws, wn, we, b2d)

    out = pl.pallas_call(
        _p3_body,
        grid=(N // _T3,),
        in_specs=[
            pl.BlockSpec((8, A), lambda i: (0, 0)),
            pl.BlockSpec((1, A), lambda i: (0, 0)),
            pl.BlockSpec((1, A), lambda i: (0, 0)),
            pl.BlockSpec((_T3, A), lambda i: (i, 0)),
            pl.BlockSpec((_T3, A), lambda i: (i, 0)),
        ],
        out_specs=pl.BlockSpec((_T3, A), lambda i: (i, 0)),
        out_shape=jax.ShapeDtypeStruct((N, A), jnp.float32),
    )(st2, g2, be2, atom_in_fea, ns)

    return out


# speedup vs baseline: 2.6226x; 1.2050x over previous
import functools

import jax
import jax.numpy as jnp
from jax import lax
from jax.experimental import pallas as pl
from jax.experimental.pallas import tpu as pltpu
from jax.experimental.pallas import tpu_sc as plsc

A = 128
NBR = 16
N = 10000
M = 32
EPS = 1e-5

E = N * M
_NS = 16
_CW = 128
_ROWS = E // _CW
_RPW = 160
_RPL = _ROWS - 15 * _RPW

_T = 200
_TE = _T * M
_GRID = N // _T
_T3 = 2000

_NB = 6
_KL = 3


def _sc_gather(table, idx2d):
    mesh = plsc.VectorSubcoreMesh(core_axis_name="c", subcore_axis_name="s",
                                  num_cores=1)

    @functools.partial(
        pl.kernel,
        out_type=jax.ShapeDtypeStruct((_ROWS, _CW, A), jnp.float32),
        mesh=mesh,
        scratch_types=[
            pltpu.VMEM((_RPW, _CW), jnp.int32),
            pltpu.VMEM((_NB * _CW, A), jnp.float32),
            pltpu.SemaphoreType.DMA((_NB,)),
        ],
    )
    def k(table_hbm, idx_hbm, out_hbm, idx_v, bufs, sems):
        sid = lax.axis_index("s")
        rbase = sid * _RPW
        nrows = jnp.where(sid == _NS - 1, _RPL, _RPW)

        @pl.when(sid != _NS - 1)
        def _():
            pltpu.sync_copy(idx_hbm.at[pl.ds(rbase, _RPW)], idx_v)

        @pl.when(sid == _NS - 1)
        def _():
            pltpu.sync_copy(idx_hbm.at[pl.ds(rbase, 96)],
                            idx_v.at[pl.ds(0, 96)])
            pltpu.sync_copy(idx_hbm.at[pl.ds(rbase + 96, 4)],
                            idx_v.at[pl.ds(96, 4)])

        def body(jj, carry):
            b = lax.rem(jj, _NB)
            buf_b = bufs.at[pl.ds(b * _CW, _CW)]

            @pl.when((jj >= _NB) & (jj - _NB < nrows - (_NB - _KL)))
            def _():
                pltpu.make_async_copy(
                    buf_b, out_hbm.at[rbase + jj - _NB], sems.at[b]).wait()

            @pl.when(jj < nrows)
            def _():
                pltpu.async_copy(
                    table_hbm.at[idx_v.at[jj]], buf_b, sems.at[b])

            j2 = jj - _KL
            b2 = lax.rem(j2 + _NB, _NB)
            buf_b2 = bufs.at[pl.ds(b2 * _CW, _CW)]

            @pl.when((jj >= _KL) & (j2 < nrows))
            def _():
                pltpu.make_async_copy(
                    table_hbm.at[idx_v.at[0]], buf_b2, sems.at[b2]).wait()
                pltpu.async_copy(buf_b2, out_hbm.at[rbase + j2], sems.at[b2])

            return carry

        lax.fori_loop(0, _RPW + _KL, body, 0)

        def drain(t, carry):
            c2 = nrows - (_NB - _KL) + t
            b = lax.rem(c2, _NB)
            pltpu.make_async_copy(
                bufs.at[pl.ds(b * _CW, _CW)],
                out_hbm.at[rbase + c2], sems.at[b]).wait()
            return carry

        lax.fori_loop(0, _NB - _KL, drain, 0)

    return k(table, idx2d)


def _p1_body(staged_ref, nbr_ref, atom_ref, ws_ref, wn_ref, we_ref, b_ref,
             out_ref):
    i = pl.program_id(0)
    xg = staged_ref[...].astype(jnp.bfloat16)
    z = (jnp.dot(xg, wn_ref[...], preferred_element_type=jnp.float32)
         + jnp.dot(nbr_ref[...], we_ref[...], preferred_element_type=jnp.float32))
    s = jnp.dot(atom_ref[...], ws_ref[...], preferred_element_type=jnp.float32) + b_ref[...]
    z3 = z.reshape(_T, M, 2 * A) + s[:, None, :]

    @pl.when(i == 0)
    def _():
        out_ref[...] = jnp.zeros_like(out_ref)

    out_ref[0:1, :] += jnp.sum(z3, axis=(0, 1))[None, :]
    out_ref[1:2, :] += jnp.sum(z3 * z3, axis=(0, 1))[None, :]


def _p2_body(sums_ref, g1_ref, b1_ref, staged_ref, nbr_ref, atom_ref,
             ws_ref, wn_ref, we_ref, b_ref, ns_ref, st2_ref):
    i = pl.program_id(0)
    nm = jnp.float32(E)
    mean = sums_ref[0:1, :] / nm
    var = sums_ref[1:2, :] / nm - mean * mean
    a = g1_ref[...] * lax.rsqrt(var + EPS)
    d = b1_ref[...] - mean * a

    xg = staged_ref[...].astype(jnp.bfloat16)
    z = (jnp.dot(xg, wn_ref[...], preferred_element_type=jnp.float32)
         + jnp.dot(nbr_ref[...], we_ref[...], preferred_element_type=jnp.float32))
    s = jnp.dot(atom_ref[...], ws_ref[...], preferred_element_type=jnp.float32) + b_ref[...]
    z3 = z.reshape(_T, M, 2 * A) + s[:, None, :]
    zt = z3 * a[0][None, None, :] + d[0][None, None, :]

    f = zt[:, :, :A]
    c = zt[:, :, A:]
    p = (1.0 / (1.0 + jnp.exp(-f))) * jnp.maximum(c, 0.0)
    ns = jnp.sum(p, axis=1)
    ns_ref[...] = ns

    @pl.when(i == 0)
    def _():
        st2_ref[...] = jnp.zeros_like(st2_ref)

    st2_ref[0:1, :] += jnp.sum(ns, axis=0)[None, :]
    st2_ref[1:2, :] += jnp.sum(ns * ns, axis=0)[None, :]


def _p3_body(st2_ref, g2_ref, b2_ref, atom_ref, ns_ref, out_ref):
    nn = jnp.float32(N)
    mean = st2_ref[0:1, :] / nn
    var = st2_ref[1:2, :] / nn - mean * mean
    a = g2_ref[...] * lax.rsqrt(var + EPS)
    d = b2_ref[...] - mean * a
    out_ref[...] = jnp.maximum(atom_ref[...] + ns_ref[...] * a + d, 0.0)


def kernel(atom_in_fea, nbr_fea, nbr_fea_idx, W_full, b_full,
           bn1_gamma, bn1_beta, bn2_gamma, bn2_beta):
    atom_in_fea = atom_in_fea.astype(jnp.float32)
    idx2d = nbr_fea_idx.astype(jnp.int32).reshape(_ROWS, _CW)

    staged = _sc_gather(atom_in_fea, idx2d).reshape(E, A)
    nbr2 = nbr_fea.astype(jnp.float32).reshape(E, NBR)

    ws = W_full[:A]
    wn = W_full[A:2 * A].astype(jnp.bfloat16)
    we = W_full[2 * A:]
    b2d = b_full.reshape(1, 2 * A)
    g1 = bn1_gamma.reshape(1, 2 * A)
    be1 = bn1_beta.reshape(1, 2 * A)
    g2 = bn2_gamma.reshape(1, A)
    be2 = bn2_beta.reshape(1, A)

    edge_specs = [
        pl.BlockSpec((_TE, A), lambda i: (i, 0)),
        pl.BlockSpec((_TE, NBR), lambda i: (i, 0)),
        pl.BlockSpec((_T, A), lambda i: (i, 0)),
        pl.BlockSpec((A, 2 * A), lambda i: (0, 0)),
        pl.BlockSpec((A, 2 * A), lambda i: (0, 0)),
        pl.BlockSpec((NBR, 2 * A), lambda i: (0, 0)),
        pl.BlockSpec((1, 2 * A), lambda i: (0, 0)),
    ]

    sums = pl.pallas_call(
        _p1_body,
        grid=(_GRID,),
        in_specs=edge_specs,
        out_specs=pl.BlockSpec((8, 2 * A), lambda i: (0, 0)),
        out_shape=jax.ShapeDtypeStruct((8, 2 * A), jnp.float32),
        compiler_params=pltpu.CompilerParams(
            dimension_semantics=("arbitrary",)),
    )(staged, nbr2, atom_in_fea, ws, wn, we, b2d)

    small = [
        pl.BlockSpec((8, 2 * A), lambda i: (0, 0)),
        pl.BlockSpec((1, 2 * A), lambda i: (0, 0)),
        pl.BlockSpec((1, 2 * A), lambda i: (0, 0)),
    ]
    ns, st2 = pl.pallas_call(
        _p2_body,
        grid=(_GRID,),
        in_specs=small + edge_specs,
        out_specs=[
            pl.BlockSpec((_T, A), lambda i: (i, 0)),
            pl.BlockSpec((8, A), lambda i: (0, 0)),
        ],
        out_shape=[
            jax.ShapeDtypeStruct((N, A), jnp.float32),
            jax.ShapeDtypeStruct((8, A), jnp.float32),
        ],
        compiler_params=pltpu.CompilerParams(
            dimension_semantics=("arbitrary",)),
    )(sums, g1, be1, staged, nbr2, atom_in_fea, ws, wn, we, b2d)

    out = pl.pallas_call(
        _p3_body,
        grid=(N // _T3,),
        in_specs=[
            pl.BlockSpec((8, A), lambda i: (0, 0)),
            pl.BlockSpec((1, A), lambda i: (0, 0)),
            pl.BlockSpec((1, A), lambda i: (0, 0)),
            pl.BlockSpec((_T3, A), lambda i: (i, 0)),
            pl.BlockSpec((_T3, A), lambda i: (i, 0)),
        ],
        out_specs=pl.BlockSpec((_T3, A), lambda i: (i, 0)),
        out_shape=jax.ShapeDtypeStruct((N, A), jnp.float32),
    )(st2, g2, be2, atom_in_fea, ns)

    return out


# speedup vs baseline: 2.6496x; 1.0103x over previous
import functools

import jax
import jax.numpy as jnp
from jax import lax
from jax.experimental import pallas as pl
from jax.experimental.pallas import tpu as pltpu
from jax.experimental.pallas import tpu_sc as plsc

A = 128
NBR = 16
N = 10000
M = 32
EPS = 1e-5

E = N * M
_NS = 16
_CW = 128
_ROWS = E // _CW
_RPW = 80
_RA = 1200
_RB = 1300

_T = 200
_TE = _T * M
_GRID = N // _T
_GA = _RA * _CW // _TE
_GB = _RB * _CW // _TE
_T3 = 2000

_NB = 6
_KL = 3


def _sc_gather(table, idx2d, row_off, nrows_half, rpw_last):
    mesh = plsc.VectorSubcoreMesh(core_axis_name="c", subcore_axis_name="s",
                                  num_cores=1)

    @functools.partial(
        pl.kernel,
        out_type=jax.ShapeDtypeStruct((nrows_half, _CW, A), jnp.float32),
        mesh=mesh,
        scratch_types=[
            pltpu.VMEM((max(_RPW, rpw_last), _CW), jnp.int32),
            pltpu.VMEM((_NB * _CW, A), jnp.float32),
            pltpu.SemaphoreType.DMA((_NB,)),
        ],
    )
    def k(table_hbm, idx_hbm, out_hbm, idx_v, bufs, sems):
        sid = lax.axis_index("s")
        rbase = row_off + sid * _RPW
        obase = sid * _RPW
        nrows = jnp.where(sid == _NS - 1, rpw_last, _RPW)

        @pl.when(sid != _NS - 1)
        def _():
            pltpu.sync_copy(idx_hbm.at[pl.ds(rbase, _RPW)],
                            idx_v.at[pl.ds(0, _RPW)])

        if rpw_last > 0:
            @pl.when(sid == _NS - 1)
            def _():
                pltpu.sync_copy(idx_hbm.at[pl.ds(rbase, 96)],
                                idx_v.at[pl.ds(0, 96)])
                pltpu.sync_copy(idx_hbm.at[pl.ds(rbase + 96, 4)],
                                idx_v.at[pl.ds(96, 4)])

        def body(jj, carry):
            b = lax.rem(jj, _NB)
            buf_b = bufs.at[pl.ds(b * _CW, _CW)]

            @pl.when((jj >= _NB) & (jj - _NB < nrows - (_NB - _KL)))
            def _():
                pltpu.make_async_copy(
                    buf_b, out_hbm.at[obase + jj - _NB], sems.at[b]).wait()

            @pl.when(jj < nrows)
            def _():
                pltpu.async_copy(
                    table_hbm.at[idx_v.at[jj]], buf_b, sems.at[b])

            j2 = jj - _KL
            b2 = lax.rem(j2 + _NB, _NB)
            buf_b2 = bufs.at[pl.ds(b2 * _CW, _CW)]

            @pl.when((jj >= _KL) & (j2 < nrows))
            def _():
                pltpu.make_async_copy(
                    table_hbm.at[idx_v.at[0]], buf_b2, sems.at[b2]).wait()
                pltpu.async_copy(buf_b2, out_hbm.at[obase + j2], sems.at[b2])

            return carry

        lax.fori_loop(0, max(_RPW, rpw_last) + _KL, body, 0)

        def drain(t, carry):
            c2 = nrows - (_NB - _KL) + t

            @pl.when(c2 >= 0)
            def _():
                b = lax.rem(c2 + _NB, _NB)
                pltpu.make_async_copy(
                    bufs.at[pl.ds(b * _CW, _CW)],
                    out_hbm.at[obase + c2], sems.at[b]).wait()
            return carry

        lax.fori_loop(0, _NB - _KL, drain, 0)

    return k(table, idx2d)


def _p1_body(staged_ref, nbr_ref, atom_ref, ws_ref, wn_ref, we_ref, b_ref,
             out_ref):
    i = pl.program_id(0)
    xg = staged_ref[...].astype(jnp.bfloat16)
    z = (jnp.dot(xg, wn_ref[...], preferred_element_type=jnp.float32)
         + jnp.dot(nbr_ref[...], we_ref[...], preferred_element_type=jnp.float32))
    s = jnp.dot(atom_ref[...], ws_ref[...], preferred_element_type=jnp.float32) + b_ref[...]
    z3 = z.reshape(_T, M, 2 * A) + s[:, None, :]

    @pl.when(i == 0)
    def _():
        out_ref[...] = jnp.zeros_like(out_ref)

    out_ref[0:1, :] += jnp.sum(z3, axis=(0, 1))[None, :]
    out_ref[1:2, :] += jnp.sum(z3 * z3, axis=(0, 1))[None, :]


def _p2_body(sums_ref, g1_ref, b1_ref, staged_ref, nbr_ref, atom_ref,
             ws_ref, wn_ref, we_ref, b_ref, ns_ref, st2_ref):
    i = pl.program_id(0)
    nm = jnp.float32(E)
    mean = sums_ref[0:1, :] / nm
    var = sums_ref[1:2, :] / nm - mean * mean
    a = g1_ref[...] * lax.rsqrt(var + EPS)
    d = b1_ref[...] - mean * a

    xg = staged_ref[...].astype(jnp.bfloat16)
    z = (jnp.dot(xg, wn_ref[...], preferred_element_type=jnp.float32)
         + jnp.dot(nbr_ref[...], we_ref[...], preferred_element_type=jnp.float32))
    s = jnp.dot(atom_ref[...], ws_ref[...], preferred_element_type=jnp.float32) + b_ref[...]
    z3 = z.reshape(_T, M, 2 * A) + s[:, None, :]
    zt = z3 * a[0][None, None, :] + d[0][None, None, :]

    f = zt[:, :, :A]
    c = zt[:, :, A:]
    p = (1.0 / (1.0 + jnp.exp(-f))) * jnp.maximum(c, 0.0)
    ns = jnp.sum(p, axis=1)
    ns_ref[...] = ns

    @pl.when(i == 0)
    def _():
        st2_ref[...] = jnp.zeros_like(st2_ref)

    st2_ref[0:1, :] += jnp.sum(ns, axis=0)[None, :]
    st2_ref[1:2, :] += jnp.sum(ns * ns, axis=0)[None, :]


def _p3_body(st2_ref, g2_ref, b2_ref, atom_ref, ns_ref, out_ref):
    nn = jnp.float32(N)
    mean = st2_ref[0:1, :] / nn
    var = st2_ref[1:2, :] / nn - mean * mean
    a = g2_ref[...] * lax.rsqrt(var + EPS)
    d = b2_ref[...] - mean * a
    out_ref[...] = jnp.maximum(atom_ref[...] + ns_ref[...] * a + d, 0.0)


def kernel(atom_in_fea, nbr_fea, nbr_fea_idx, W_full, b_full,
           bn1_gamma, bn1_beta, bn2_gamma, bn2_beta):
    atom_in_fea = atom_in_fea.astype(jnp.float32)
    idx2d = nbr_fea_idx.astype(jnp.int32).reshape(_ROWS, _CW)

    staged_a = _sc_gather(atom_in_fea, idx2d, 0, _RA, 0).reshape(_RA * _CW, A)
    staged_b = _sc_gather(atom_in_fea, idx2d, _RA, _RB, 100).reshape(_RB * _CW, A)
    nbr2 = nbr_fea.astype(jnp.float32).reshape(E, NBR)

    ws = W_full[:A]
    wn = W_full[A:2 * A].astype(jnp.bfloat16)
    we = W_full[2 * A:]
    b2d = b_full.reshape(1, 2 * A)
    g1 = bn1_gamma.reshape(1, 2 * A)
    be1 = bn1_beta.reshape(1, 2 * A)
    g2 = bn2_gamma.reshape(1, A)
    be2 = bn2_beta.reshape(1, A)

    def edge_specs(off):
        return [
            pl.BlockSpec((_TE, A), lambda i: (i, 0)),
            pl.BlockSpec((_TE, NBR), lambda i: (i + off, 0)),
            pl.BlockSpec((_T, A), lambda i: (i + off, 0)),
            pl.BlockSpec((A, 2 * A), lambda i: (0, 0)),
            pl.BlockSpec((A, 2 * A), lambda i: (0, 0)),
            pl.BlockSpec((NBR, 2 * A), lambda i: (0, 0)),
            pl.BlockSpec((1, 2 * A), lambda i: (0, 0)),
        ]

    def p1(staged, grid, off):
        return pl.pallas_call(
            _p1_body,
            grid=(grid,),
            in_specs=edge_specs(off),
            out_specs=pl.BlockSpec((8, 2 * A), lambda i: (0, 0)),
            out_shape=jax.ShapeDtypeStruct((8, 2 * A), jnp.float32),
            compiler_params=pltpu.CompilerParams(
                dimension_semantics=("arbitrary",)),
        )(staged, nbr2, atom_in_fea, ws, wn, we, b2d)

    sums = p1(staged_a, _GA, 0) + p1(staged_b, _GB, _GA)

    small = [
        pl.BlockSpec((8, 2 * A), lambda i: (0, 0)),
        pl.BlockSpec((1, 2 * A), lambda i: (0, 0)),
        pl.BlockSpec((1, 2 * A), lambda i: (0, 0)),
    ]

    def p2(staged, grid, off, n_half):
        return pl.pallas_call(
            _p2_body,
            grid=(grid,),
            in_specs=small + edge_specs(off),
            out_specs=[
                pl.BlockSpec((_T, A), lambda i: (i, 0)),
                pl.BlockSpec((8, A), lambda i: (0, 0)),
            ],
            out_shape=[
                jax.ShapeDtypeStruct((n_half, A), jnp.float32),
                jax.ShapeDtypeStruct((8, A), jnp.float32),
            ],
            compiler_params=pltpu.CompilerParams(
                dimension_semantics=("arbitrary",)),
        )(sums, g1, be1, staged, nbr2, atom_in_fea, ws, wn, we, b2d)

    ns_a, st2_a = p2(staged_a, _GA, 0, _GA * _T)
    ns_b, st2_b = p2(staged_b, _GB, _GA, _GB * _T)
    st2 = st2_a + st2_b
    ns = jnp.concatenate([ns_a, ns_b], axis=0)

    out = pl.pallas_call(
        _p3_body,
        grid=(N // _T3,),
        in_specs=[
            pl.BlockSpec((8, A), lambda i: (0, 0)),
            pl.BlockSpec((1, A), lambda i: (0, 0)),
            pl.BlockSpec((1, A), lambda i: (0, 0)),
            pl.BlockSpec((_T3, A), lambda i: (i, 0)),
            pl.BlockSpec((_T3, A), lambda i: (i, 0)),
        ],
        out_specs=pl.BlockSpec((_T3, A), lambda i: (i, 0)),
        out_shape=jax.ShapeDtypeStruct((N, A), jnp.float32),
    )(st2, g2, be2, atom_in_fea, ns)

    return out


# speedup vs baseline: 2.6520x; 1.0009x over previous
import functools

import jax
import jax.numpy as jnp
from jax import lax
from jax.experimental import pallas as pl
from jax.experimental.pallas import tpu as pltpu
from jax.experimental.pallas import tpu_sc as plsc

A = 128
NBR = 16
N = 10000
M = 32
EPS = 1e-5

E = N * M
_NS = 16
_CW = 128
_ROWS = E // _CW
_RPW = 80
_RA = 1200
_RB = 1300

_T = 400
_TE = _T * M
_GRID = N // _T
_GA = _RA * _CW // _TE
_GB = _RB * _CW // _TE
_T3 = 2000

_NB = 6
_KL = 3


def _sc_gather(table, idx2d, row_off, nrows_half, rpw_last):
    mesh = plsc.VectorSubcoreMesh(core_axis_name="c", subcore_axis_name="s",
                                  num_cores=1)

    @functools.partial(
        pl.kernel,
        out_type=jax.ShapeDtypeStruct((nrows_half, _CW, A), jnp.float32),
        mesh=mesh,
        scratch_types=[
            pltpu.VMEM((max(_RPW, rpw_last), _CW), jnp.int32),
            pltpu.VMEM((_NB * _CW, A), jnp.float32),
            pltpu.SemaphoreType.DMA((_NB,)),
        ],
    )
    def k(table_hbm, idx_hbm, out_hbm, idx_v, bufs, sems):
        sid = lax.axis_index("s")
        rbase = row_off + sid * _RPW
        obase = sid * _RPW
        nrows = jnp.where(sid == _NS - 1, rpw_last, _RPW)

        @pl.when(sid != _NS - 1)
        def _():
            pltpu.sync_copy(idx_hbm.at[pl.ds(rbase, _RPW)],
                            idx_v.at[pl.ds(0, _RPW)])

        if rpw_last > 0:
            @pl.when(sid == _NS - 1)
            def _():
                pltpu.sync_copy(idx_hbm.at[pl.ds(rbase, 96)],
                                idx_v.at[pl.ds(0, 96)])
                pltpu.sync_copy(idx_hbm.at[pl.ds(rbase + 96, 4)],
                                idx_v.at[pl.ds(96, 4)])

        def body(jj, carry):
            b = lax.rem(jj, _NB)
            buf_b = bufs.at[pl.ds(b * _CW, _CW)]

            @pl.when((jj >= _NB) & (jj - _NB < nrows - (_NB - _KL)))
            def _():
                pltpu.make_async_copy(
                    buf_b, out_hbm.at[obase + jj - _NB], sems.at[b]).wait()

            @pl.when(jj < nrows)
            def _():
                pltpu.async_copy(
                    table_hbm.at[idx_v.at[jj]], buf_b, sems.at[b])

            j2 = jj - _KL
            b2 = lax.rem(j2 + _NB, _NB)
            buf_b2 = bufs.at[pl.ds(b2 * _CW, _CW)]

            @pl.when((jj >= _KL) & (j2 < nrows))
            def _():
                pltpu.make_async_copy(
                    table_hbm.at[idx_v.at[0]], buf_b2, sems.at[b2]).wait()
                pltpu.async_copy(buf_b2, out_hbm.at[obase + j2], sems.at[b2])

            return carry

        lax.fori_loop(0, max(_RPW, rpw_last) + _KL, body, 0)

        def drain(t, carry):
            c2 = nrows - (_NB - _KL) + t

            @pl.when(c2 >= 0)
            def _():
                b = lax.rem(c2 + _NB, _NB)
                pltpu.make_async_copy(
                    bufs.at[pl.ds(b * _CW, _CW)],
                    out_hbm.at[obase + c2], sems.at[b]).wait()
            return carry

        lax.fori_loop(0, _NB - _KL, drain, 0)

    return k(table, idx2d)


def _p1_body(staged_ref, nbr_ref, atom_ref, ws_ref, wn_ref, we_ref, b_ref,
             out_ref):
    i = pl.program_id(0)
    xg = staged_ref[...].astype(jnp.bfloat16)
    z = (jnp.dot(xg, wn_ref[...], preferred_element_type=jnp.float32)
         + jnp.dot(nbr_ref[...], we_ref[...], preferred_element_type=jnp.float32))
    s = jnp.dot(atom_ref[...], ws_ref[...], preferred_element_type=jnp.float32) + b_ref[...]
    z3 = z.reshape(_T, M, 2 * A) + s[:, None, :]

    @pl.when(i == 0)
    def _():
        out_ref[...] = jnp.zeros_like(out_ref)

    out_ref[0:1, :] += jnp.sum(z3, axis=(0, 1))[None, :]
    out_ref[1:2, :] += jnp.sum(z3 * z3, axis=(0, 1))[None, :]


def _p2_body(sums_ref, g1_ref, b1_ref, staged_ref, nbr_ref, atom_ref,
             ws_ref, wn_ref, we_ref, b_ref, ns_ref, st2_ref):
    i = pl.program_id(0)
    nm = jnp.float32(E)
    mean = sums_ref[0:1, :] / nm
    var = sums_ref[1:2, :] / nm - mean * mean
    a = g1_ref[...] * lax.rsqrt(var + EPS)
    d = b1_ref[...] - mean * a

    xg = staged_ref[...].astype(jnp.bfloat16)
    z = (jnp.dot(xg, wn_ref[...], preferred_element_type=jnp.float32)
         + jnp.dot(nbr_ref[...], we_ref[...], preferred_element_type=jnp.float32))
    s = jnp.dot(atom_ref[...], ws_ref[...], preferred_element_type=jnp.float32) + b_ref[...]
    z3 = z.reshape(_T, M, 2 * A) + s[:, None, :]
    zt = z3 * a[0][None, None, :] + d[0][None, None, :]

    f = zt[:, :, :A]
    c = zt[:, :, A:]
    p = (1.0 / (1.0 + jnp.exp(-f))) * jnp.maximum(c, 0.0)
    ns = jnp.sum(p, axis=1)
    ns_ref[...] = ns

    @pl.when(i == 0)
    def _():
        st2_ref[...] = jnp.zeros_like(st2_ref)

    st2_ref[0:1, :] += jnp.sum(ns, axis=0)[None, :]
    st2_ref[1:2, :] += jnp.sum(ns * ns, axis=0)[None, :]


def _p3_body(st2_ref, g2_ref, b2_ref, atom_ref, ns_ref, out_ref):
    nn = jnp.float32(N)
    mean = st2_ref[0:1, :] / nn
    var = st2_ref[1:2, :] / nn - mean * mean
    a = g2_ref[...] * lax.rsqrt(var + EPS)
    d = b2_ref[...] - mean * a
    out_ref[...] = jnp.maximum(atom_ref[...] + ns_ref[...] * a + d, 0.0)


def kernel(atom_in_fea, nbr_fea, nbr_fea_idx, W_full, b_full,
           bn1_gamma, bn1_beta, bn2_gamma, bn2_beta):
    atom_in_fea = atom_in_fea.astype(jnp.float32)
    idx2d = nbr_fea_idx.astype(jnp.int32).reshape(_ROWS, _CW)

    staged_a = _sc_gather(atom_in_fea, idx2d, 0, _RA, 0).reshape(_RA * _CW, A)
    staged_b = _sc_gather(atom_in_fea, idx2d, _RA, _RB, 100).reshape(_RB * _CW, A)
    nbr2 = nbr_fea.astype(jnp.float32).reshape(E, NBR)

    ws = W_full[:A]
    wn = W_full[A:2 * A].astype(jnp.bfloat16)
    we = W_full[2 * A:]
    b2d = b_full.reshape(1, 2 * A)
    g1 = bn1_gamma.reshape(1, 2 * A)
    be1 = bn1_beta.reshape(1, 2 * A)
    g2 = bn2_gamma.reshape(1, A)
    be2 = bn2_beta.reshape(1, A)

    def edge_specs(off):
        return [
            pl.BlockSpec((_TE, A), lambda i: (i, 0)),
            pl.BlockSpec((_TE, NBR), lambda i: (i + off, 0)),
            pl.BlockSpec((_T, A), lambda i: (i + off, 0)),
            pl.BlockSpec((A, 2 * A), lambda i: (0, 0)),
            pl.BlockSpec((A, 2 * A), lambda i: (0, 0)),
            pl.BlockSpec((NBR, 2 * A), lambda i: (0, 0)),
            pl.BlockSpec((1, 2 * A), lambda i: (0, 0)),
        ]

    def p1(staged, grid, off):
        return pl.pallas_call(
            _p1_body,
            grid=(grid,),
            in_specs=edge_specs(off),
            out_specs=pl.BlockSpec((8, 2 * A), lambda i: (0, 0)),
            out_shape=jax.ShapeDtypeStruct((8, 2 * A), jnp.float32),
            compiler_params=pltpu.CompilerParams(
                dimension_semantics=("arbitrary",)),
        )(staged, nbr2, atom_in_fea, ws, wn, we, b2d)

    sums = p1(staged_a, _GA, 0) + p1(staged_b, _GB, _GA)

    small = [
        pl.BlockSpec((8, 2 * A), lambda i: (0, 0)),
        pl.BlockSpec((1, 2 * A), lambda i: (0, 0)),
        pl.BlockSpec((1, 2 * A), lambda i: (0, 0)),
    ]

    def p2(staged, grid, off, n_half):
        return pl.pallas_call(
            _p2_body,
            grid=(grid,),
            in_specs=small + edge_specs(off),
            out_specs=[
                pl.BlockSpec((_T, A), lambda i: (i, 0)),
                pl.BlockSpec((8, A), lambda i: (0, 0)),
            ],
            out_shape=[
                jax.ShapeDtypeStruct((n_half, A), jnp.float32),
                jax.ShapeDtypeStruct((8, A), jnp.float32),
            ],
            compiler_params=pltpu.CompilerParams(
                dimension_semantics=("arbitrary",)),
        )(sums, g1, be1, staged, nbr2, atom_in_fea, ws, wn, we, b2d)

    ns_a, st2_a = p2(staged_a, _GA, 0, _GA * _T)
    ns_b, st2_b = p2(staged_b, _GB, _GA, _GB * _T)
    st2 = st2_a + st2_b
    ns = jnp.concatenate([ns_a, ns_b], axis=0)

    out = pl.pallas_call(
        _p3_body,
        grid=(N // _T3,),
        in_specs=[
            pl.BlockSpec((8, A), lambda i: (0, 0)),
            pl.BlockSpec((1, A), lambda i: (0, 0)),
            pl.BlockSpec((1, A), lambda i: (0, 0)),
            pl.BlockSpec((_T3, A), lambda i: (i, 0)),
            pl.BlockSpec((_T3, A), lambda i: (i, 0)),
        ],
        out_specs=pl.BlockSpec((_T3, A), lambda i: (i, 0)),
        out_shape=jax.ShapeDtypeStruct((N, A), jnp.float32),
    )(st2, g2, be2, atom_in_fea, ns)

    return out
